# baseline (speedup 1.0000x reference)
import functools

import jax
import jax.numpy as jnp
from jax import lax
from jax.experimental import pallas as pl
from jax.experimental.pallas import tpu as pltpu
from jax.experimental.pallas import tpu_sc as plsc

B = 16384
D = 32
NC = 2
NS = 16
NW = NC * NS
BPW = B // NW

_MESH = plsc.VectorSubcoreMesh(core_axis_name="c", subcore_axis_name="s")


@functools.partial(
    pl.kernel,
    mesh=_MESH,
    out_type=[jax.ShapeDtypeStruct((B, D), jnp.float32)] * 4,
    scratch_types=[
        pltpu.VMEM((BPW,), jnp.int32),
        pltpu.VMEM((BPW,), jnp.int32),
        pltpu.VMEM((BPW, D), jnp.float32),
        pltpu.VMEM((BPW, D), jnp.float32),
        pltpu.VMEM((BPW, D), jnp.float32),
        pltpu.VMEM((BPW, D), jnp.float32),
        pltpu.SemaphoreType.DMA,
    ],
    compiler_params=pltpu.CompilerParams(use_tc_tiling_on_sc=False),
)
def _sc_gather(u_hbm, i_hbm, ug_hbm, ig_hbm, um_hbm, im_hbm,
               gu_out, gi_out, hu_out, hi_out,
               u_v, i_v, a_v, b_v, c_v, d_v, sem):
    wid = lax.axis_index("s") * NC + lax.axis_index("c")
    base = wid * BPW
    pltpu.sync_copy(u_hbm.at[pl.ds(base, BPW)], u_v)
    pltpu.sync_copy(i_hbm.at[pl.ds(base, BPW)], i_v)
    cp1 = pltpu.async_copy(ug_hbm.at[u_v], a_v, sem)
    cp2 = pltpu.async_copy(ig_hbm.at[i_v], b_v, sem)
    cp3 = pltpu.async_copy(um_hbm.at[u_v], c_v, sem)
    cp4 = pltpu.async_copy(im_hbm.at[i_v], d_v, sem)
    cp1.wait()
    cp2.wait()
    cp3.wait()
    cp4.wait()
    pltpu.sync_copy(a_v, gu_out.at[pl.ds(base, BPW)])
    pltpu.sync_copy(b_v, gi_out.at[pl.ds(base, BPW)])
    pltpu.sync_copy(c_v, hu_out.at[pl.ds(base, BPW)])
    pltpu.sync_copy(d_v, hi_out.at[pl.ds(base, BPW)])


BLK = 2048


def _mlp_body(gu, gi, hu, hi, w1u, w1i, b1, w2, b2, w3, b3, wfg, wfh, bf,
              out):
    g = gu[...] * gi[...]
    h = jnp.dot(hu[...], w1u[...], preferred_element_type=jnp.float32)
    h = h + jnp.dot(hi[...], w1i[...], preferred_element_type=jnp.float32)
    h = jnp.maximum(h + b1[...], 0.0)
    h = jnp.maximum(
        jnp.dot(h, w2[...], preferred_element_type=jnp.float32) + b2[...], 0.0)
    h = jnp.maximum(
        jnp.dot(h, w3[...], preferred_element_type=jnp.float32) + b3[...], 0.0)
    out[...] = (jnp.dot(g, wfg[...], preferred_element_type=jnp.float32)
                + jnp.dot(h, wfh[...], preferred_element_type=jnp.float32)
                + bf[...])


def _mlp(gu, gi, hu, hi, W1u, W1i, b1, W2, b2, W3, b3, Wfg, Wfh, bf):
    grid = (B // BLK,)
    row_spec = pl.BlockSpec((BLK, D), lambda j: (j, 0))
    full = lambda s: pl.BlockSpec(s, lambda j: (0,) * len(s))
    return pl.pallas_call(
        _mlp_body,
        grid=grid,
        in_specs=[
            row_spec, row_spec, row_spec, row_spec,
            full((D, 64)), full((D, 64)), full((1, 64)),
            full((64, 32)), full((1, 32)),
            full((32, 16)), full((1, 16)),
            full((D, 1)), full((16, 1)), full((1, 1)),
        ],
        out_specs=pl.BlockSpec((BLK, 1), lambda j: (j, 0)),
        out_shape=jax.ShapeDtypeStruct((B, 1), jnp.float32),
    )(gu, gi, hu, hi, W1u, W1i, b1, W2, b2, W3, b3, Wfg, Wfh, bf)


def kernel(u, i, ug, ig, um, im, W1, b1, W2, b2, W3, b3, Wf, bf):
    u = u.astype(jnp.int32)
    i = i.astype(jnp.int32)
    gu, gi, hu, hi = _sc_gather(u, i, ug, ig, um, im)
    out = _mlp(gu, gi, hu, hi,
               W1[:D], W1[D:], b1.reshape(1, 64),
               W2, b2.reshape(1, 32), W3, b3.reshape(1, 16),
               Wf[:D], Wf[D:], bf.reshape(1, 1))
    return out.reshape(-1)
